# baseline (speedup 1.0000x reference)
import functools

import jax
import jax.numpy as jnp
from jax.experimental import pallas as pl


def _ea_all_layers(edge_attr, Wcat):
    E, LF = edge_attr.shape
    K = Wcat.shape[1]
    B = 8000
    assert E % B == 0

    def body(a_ref, w_ref, o_ref):
        o_ref[...] = jnp.dot(a_ref[...], w_ref[...],
                             preferred_element_type=jnp.float32)

    return pl.pallas_call(
        body,
        grid=(E // B,),
        in_specs=[pl.BlockSpec((B, LF), lambda i: (i, 0)),
                  pl.BlockSpec((LF, K), lambda i: (0, 0))],
        out_specs=pl.BlockSpec((B, K), lambda i: (i, 0)),
        out_shape=jax.ShapeDtypeStruct((E, K), jnp.float32),
    )(edge_attr, Wcat)


def _gatv2_layer(x, src, dst, ea, p, n):
    xl = x @ p["Wl"]
    xr = x @ p["Wr"]
    m = xl[src] + xr[dst] + ea
    m = jax.nn.leaky_relu(m, negative_slope=0.2)
    logits = jnp.sum(m * p["att"], axis=-1)
    mx = jax.ops.segment_max(logits, dst, num_segments=n)
    mx = jnp.where(jnp.isfinite(mx), mx, 0.0)
    a = jnp.exp(logits - mx[dst])
    den = jax.ops.segment_sum(a, dst, num_segments=n)
    alpha = a / (den[dst] + 1e-16)
    out = jax.ops.segment_sum(alpha[:, None] * xl[src], dst, num_segments=n)
    return out + p["b"]


def kernel(x, edge_index, edge_attr, params):
    n = x.shape[0]
    src = edge_index[0].astype(jnp.int32)
    dst = edge_index[1].astype(jnp.int32)

    Wcat = jnp.concatenate([p["We"] for p in params], axis=1)
    EA = _ea_all_layers(edge_attr, Wcat)

    offs = []
    o = 0
    for p in params:
        offs.append(o)
        o += p["We"].shape[1]

    h = x
    for i, p in enumerate(params):
        ea = EA[:, offs[i]:offs[i] + p["We"].shape[1]]
        h = _gatv2_layer(h, src, dst, ea, p, n)
        if i not in (2, 5):
            h = jax.nn.relu(h)
    return h


# speedup vs baseline: 11.1903x; 11.1903x over previous
import functools

import jax
import jax.numpy as jnp
from jax import lax
from jax.experimental import pallas as pl
from jax.experimental.pallas import tpu as pltpu
from jax.experimental.pallas import tpu_sc as plsc

N = 100000
E = 1600000
NC = 2
NS = 16
NW = NC * NS
EW = E // NW
C = 128
NCH = EW // C
CT = EW - NCH * C
NPAD = 100096
RPT = NPAD // NS
RZ = 368
NZ = RPT // RZ
W = 16

_DOUTS = [8, 8, 2, 8, 8, 15]

_CPARAMS = pltpu.CompilerParams(use_tc_tiling_on_sc=False)


def _pad16(m):
    return jnp.zeros((m.shape[0], W), m.dtype).at[:, :m.shape[1]].set(m)


def _make_sc_layer(dout):
    def body(src_ref, dst_ref, ea_ref, xl_ref, xr_ref, z_ref, att_ref,
             part_ref, gmax_ref, lg_ref,
             lgb, srcb, dstb, srcb_t, dstb_t, eab, xlb, xrb, rowsb,
             attb, maxb, mxallb, bounce, acc, smx):
        cid = lax.axis_index("c")
        sid = lax.axis_index("s")
        wid = cid * NS + sid
        tbase = wid * EW
        it = lax.iota(jnp.int32, 16)

        pltpu.sync_copy(z_ref, acc.at[pl.ds(sid * RPT, RPT)])

        pltpu.sync_copy(att_ref, attb)
        attv = attb[...]

        def logits_chunk(cbase, lbase, ne, sb, db):
            pltpu.sync_copy(src_ref.at[pl.ds(cbase, ne)], sb)
            pltpu.sync_copy(dst_ref.at[pl.ds(cbase, ne)], db)
            pltpu.sync_copy(ea_ref.at[pl.ds(cbase, ne)],
                            eab.at[pl.ds(0, ne)])
            pltpu.sync_copy(xl_ref.at[sb], xlb.at[pl.ds(0, ne)])
            pltpu.sync_copy(xr_ref.at[db], xrb.at[pl.ds(0, ne)])

            def grp(g, mx):
                lgv = jnp.zeros((16,), jnp.float32)
                for j in range(16):
                    e = g * 16 + j
                    m = xlb[e] + xrb[e] + eab[e]
                    m = jnp.where(m >= 0.0, m, m * 0.2)
                    c = m * attv
                    s = c[0]
                    for d in range(1, dout):
                        s = s + c[d]
                    lgv = jnp.where(it == j, jnp.broadcast_to(s, (16,)), lgv)
                lgb[pl.ds(g * 16, 16)] = lgv
                return jnp.maximum(mx, lgv)

            mx = pl.loop(0, ne // 16,
                         init_carry=jnp.full((16,), -jnp.inf, jnp.float32))(
                grp)
            pltpu.sync_copy(lgb.at[pl.ds(0, ne)],
                            lg_ref.at[pl.ds(cbase, ne)])
            return mx

        @pl.loop(0, NCH, init_carry=jnp.full((16,), -jnp.inf, jnp.float32))
        def mx_a(ci, mx):
            return jnp.maximum(mx, logits_chunk(tbase + ci * C, ci * C, C,
                                                srcb, dstb))

        mxt = jnp.maximum(mx_a,
                          logits_chunk(tbase + NCH * C, NCH * C, CT,
                                       srcb_t, dstb_t))

        maxb[...] = mxt
        pltpu.sync_copy(maxb, smx.at[pl.ds(sid * 16, 16)])
        plsc.subcore_barrier()
        pltpu.sync_copy(smx, mxallb)
        gv = jnp.full((16,), -jnp.inf, jnp.float32)
        for s in range(NS):
            gv = jnp.maximum(gv, mxallb[pl.ds(s * 16, 16)])
        g = gv[0]
        for j in range(1, 16):
            g = jnp.maximum(g, gv[j])
        gs = jnp.broadcast_to(g, (16,))
        pltpu.sync_copy(maxb, gmax_ref.at[pl.ds(wid * 16, 16)])

        def accum_chunk(cbase, lbase, ne, sb, db, rb):
            pltpu.sync_copy(src_ref.at[pl.ds(cbase, ne)], sb)
            pltpu.sync_copy(dst_ref.at[pl.ds(cbase, ne)], db)
            pltpu.sync_copy(lg_ref.at[pl.ds(cbase, ne)],
                            lgb.at[pl.ds(0, ne)])
            pltpu.sync_copy(xl_ref.at[sb], xlb.at[pl.ds(0, ne)])

            @pl.loop(0, ne // 16)
            def _(g2):
                av = jnp.exp(lgb[pl.ds(g2 * 16, 16)] - gs)
                for j in range(16):
                    e = g2 * 16 + j
                    ab = jnp.broadcast_to(av[j], (16,))
                    rowsb[e] = jnp.where(it == 15, ab, xlb[e] * ab)

            pltpu.sync_copy(rb, acc.at[db], add=True)

        @pl.loop(0, NCH)
        def _(ci):
            accum_chunk(tbase + ci * C, ci * C, C, srcb, dstb, rowsb)

        accum_chunk(tbase + NCH * C, NCH * C, CT, srcb_t, dstb_t,
                    rowsb.at[pl.ds(0, CT)])

        plsc.subcore_barrier()
        row0 = sid * RPT
        for k in range(NZ):
            pltpu.sync_copy(acc.at[pl.ds(row0 + k * RZ, RZ)], bounce)
            pltpu.sync_copy(bounce, part_ref.at[cid,
                                                pl.ds(row0 + k * RZ, RZ)])

    mesh = plsc.VectorSubcoreMesh(core_axis_name="c", subcore_axis_name="s")
    return pl.kernel(
        body,
        out_type=[jax.ShapeDtypeStruct((NC, NPAD, W), jnp.float32),
                  jax.ShapeDtypeStruct((NW * 16,), jnp.float32),
                  jax.ShapeDtypeStruct((E,), jnp.float32)],
        mesh=mesh,
        compiler_params=_CPARAMS,
        scratch_types=[
            pltpu.VMEM((C,), jnp.float32),
            pltpu.VMEM((C,), jnp.int32),
            pltpu.VMEM((C,), jnp.int32),
            pltpu.VMEM((CT,), jnp.int32),
            pltpu.VMEM((CT,), jnp.int32),
            pltpu.VMEM((C, W), jnp.float32),
            pltpu.VMEM((C, W), jnp.float32),
            pltpu.VMEM((C, W), jnp.float32),
            pltpu.VMEM((C, W), jnp.float32),
            pltpu.VMEM((16,), jnp.float32),
            pltpu.VMEM((16,), jnp.float32),
            pltpu.VMEM((NS * 16,), jnp.float32),
            pltpu.VMEM((RZ, W), jnp.float32),
            pltpu.VMEM_SHARED((NPAD, W), jnp.float32),
            pltpu.VMEM_SHARED((NS * 16,), jnp.float32),
        ],
    )


_SC_LAYERS = {d: _make_sc_layer(d) for d in (8, 2, 15)}


def _ea_all_layers(edge_attr, Ws):
    LF = edge_attr.shape[1]
    B = 8000

    def bodyf(a_ref, *refs):
        w_refs = refs[:6]
        o_refs = refs[6:]
        a = a_ref[...]
        for w_ref, o_ref in zip(w_refs, o_refs):
            o_ref[...] = jnp.dot(a, w_ref[...],
                                 preferred_element_type=jnp.float32)

    return pl.pallas_call(
        bodyf,
        grid=(E // B,),
        in_specs=[pl.BlockSpec((B, LF), lambda i: (i, 0))] +
                 [pl.BlockSpec((LF, W), lambda i: (0, 0))] * 6,
        out_specs=[pl.BlockSpec((B, W), lambda i: (i, 0))] * 6,
        out_shape=[jax.ShapeDtypeStruct((E, W), jnp.float32)] * 6,
    )(edge_attr, *Ws)


def _transform(h, Wl, Wr):
    din = h.shape[1]
    B = 5000

    def bodyf(h_ref, wl_ref, wr_ref, ol_ref, or_ref):
        hv = h_ref[...]
        ol_ref[...] = jnp.dot(hv, wl_ref[...],
                              preferred_element_type=jnp.float32)
        or_ref[...] = jnp.dot(hv, wr_ref[...],
                              preferred_element_type=jnp.float32)

    return pl.pallas_call(
        bodyf,
        grid=(N // B,),
        in_specs=[pl.BlockSpec((B, din), lambda i: (i, 0)),
                  pl.BlockSpec((din, W), lambda i: (0, 0)),
                  pl.BlockSpec((din, W), lambda i: (0, 0))],
        out_specs=[pl.BlockSpec((B, W), lambda i: (i, 0)),
                   pl.BlockSpec((B, W), lambda i: (i, 0))],
        out_shape=[jax.ShapeDtypeStruct((N, W), jnp.float32),
                   jax.ShapeDtypeStruct((N, W), jnp.float32)],
    )(h, Wl, Wr)


def _combine(part, w2, bias, dout, relu, Wl=None, Wr=None):
    B = 5000
    with_mm = Wl is not None

    def bodyf(part_ref, w_ref, b_ref, *refs):
        w0 = w_ref[0, 0]
        w1 = w_ref[0, 1]
        num = part_ref[0, :, :dout] * w0 + part_ref[1, :, :dout] * w1
        den = part_ref[0, :, 15] * w0 + part_ref[1, :, 15] * w1
        h = num / (den[:, None] + 1e-16) + b_ref[0]
        if relu:
            h = jnp.maximum(h, 0.0)
        if with_mm:
            wl_ref, wr_ref, ol_ref, or_ref = refs
            ol_ref[...] = jnp.dot(h, wl_ref[...],
                                  preferred_element_type=jnp.float32)
            or_ref[...] = jnp.dot(h, wr_ref[...],
                                  preferred_element_type=jnp.float32)
        else:
            refs[0][...] = h

    in_specs = [pl.BlockSpec((NC, B, W), lambda i: (0, i, 0)),
                pl.BlockSpec((1, NC), lambda i: (0, 0)),
                pl.BlockSpec((1, dout), lambda i: (0, 0))]
    args = [part, w2, bias.reshape(1, dout)]
    if with_mm:
        in_specs += [pl.BlockSpec((dout, W), lambda i: (0, 0)),
                     pl.BlockSpec((dout, W), lambda i: (0, 0))]
        args += [Wl, Wr]
        out_specs = [pl.BlockSpec((B, W), lambda i: (i, 0)),
                     pl.BlockSpec((B, W), lambda i: (i, 0))]
        out_shape = [jax.ShapeDtypeStruct((N, W), jnp.float32),
                     jax.ShapeDtypeStruct((N, W), jnp.float32)]
    else:
        out_specs = [pl.BlockSpec((B, dout), lambda i: (i, 0))]
        out_shape = [jax.ShapeDtypeStruct((N, dout), jnp.float32)]

    return pl.pallas_call(
        bodyf,
        grid=(N // B,),
        in_specs=in_specs,
        out_specs=out_specs,
        out_shape=out_shape,
    )(*args)


def kernel(x, edge_index, edge_attr, params):
    src = edge_index[0].astype(jnp.int32)
    dst = edge_index[1].astype(jnp.int32)

    EAs = _ea_all_layers(edge_attr, [_pad16(p["We"]) for p in params])

    xl, xr = _transform(x, _pad16(params[0]["Wl"]), _pad16(params[0]["Wr"]))

    zeros = jnp.zeros((RPT, W), jnp.float32)
    out = None
    for i, p in enumerate(params):
        dout = _DOUTS[i]
        attp = jnp.zeros((16,), jnp.float32).at[:dout].set(p["att"])
        part, gmax, _ = _SC_LAYERS[dout](src, dst, EAs[i], xl, xr, zeros,
                                        attp)
        g_sc = jnp.max(gmax.reshape(NC, NS * 16), axis=1)
        w2 = jnp.exp(g_sc - jnp.max(g_sc)).reshape(1, NC)
        relu = i not in (2, 5)
        if i < 5:
            xl, xr = _combine(part, w2, p["b"], dout, relu,
                              _pad16(params[i + 1]["Wl"]),
                              _pad16(params[i + 1]["Wr"]))
        else:
            out = _combine(part, w2, p["b"], dout, relu)[0]
    return out


# speedup vs baseline: 22.5843x; 2.0182x over previous
import functools

import jax
import jax.numpy as jnp
from jax import lax
from jax.experimental import pallas as pl
from jax.experimental.pallas import tpu as pltpu
from jax.experimental.pallas import tpu_sc as plsc

N = 100000
E = 1600000
NC = 2
NS = 16
NW = NC * NS
EW = E // NW
C = 128
K = 4
G = K * C
NG = EW // G
REM = EW - NG * G
NPAD = 100096
RPT = NPAD // NS
RZ = 184
NZ = RPT // RZ
W = 16

_DOUTS = [8, 8, 2, 8, 8, 15]

_CPARAMS = pltpu.CompilerParams(use_tc_tiling_on_sc=False)


def _pad16(m):
    return jnp.zeros((m.shape[0], W), m.dtype).at[:, :m.shape[1]].set(m)


def _make_sc_layer(dout):
    def body(src_ref, dst_ref, ea_ref, xl_ref, xr_ref, z_ref, att_ref,
             part_ref, gmax_ref, lg_ref,
             srcblk, dstblk, lgblk, eab, xlb, rowsb,
             dstb0, dstb1, dstb2, dstb3, dstb_t,
             attb, maxb, mxallb, bounce, acc, smx, sem, ssem):
        cid = lax.axis_index("c")
        sid = lax.axis_index("s")
        wid = cid * NS + sid
        tbase = wid * EW
        it = lax.iota(jnp.int32, 16)
        dstbs = [dstb0, dstb1, dstb2, dstb3]

        pltpu.sync_copy(z_ref, acc.at[pl.ds(sid * RPT, RPT)])

        pltpu.sync_copy(att_ref, attb)
        attv = attb[...]

        def logits_group(gbase, sizes, mx):
            ne = sum(sizes)
            waits = [
                pltpu.async_copy(src_ref.at[pl.ds(gbase, ne)],
                                 srcblk.at[pl.ds(0, ne)], sem),
                pltpu.async_copy(dst_ref.at[pl.ds(gbase, ne)],
                                 dstblk.at[pl.ds(0, ne)], sem),
                pltpu.async_copy(ea_ref.at[pl.ds(gbase, ne)],
                                 eab.at[pl.ds(0, ne)], sem),
            ]
            for wv in waits:
                wv.wait()
            waits = []
            off = 0
            for sz in sizes:
                waits.append(pltpu.async_copy(
                    xl_ref.at[srcblk.at[pl.ds(off, sz)]],
                    xlb.at[pl.ds(off, sz)], sem))
                waits.append(pltpu.async_copy(
                    xr_ref.at[dstblk.at[pl.ds(off, sz)]],
                    rowsb.at[pl.ds(off, sz)], sem))
                off += sz
            for wv in waits:
                wv.wait()

            def grp(g2, mxc):
                lgv = jnp.zeros((16,), jnp.float32)
                for j in range(16):
                    e = g2 * 16 + j
                    m = xlb[e] + rowsb[e] + eab[e]
                    m = jnp.where(m >= 0.0, m, m * 0.2)
                    c = m * attv
                    s = c[0]
                    for d in range(1, dout):
                        s = s + c[d]
                    lgv = jnp.where(it == j, jnp.broadcast_to(s, (16,)), lgv)
                lgblk[pl.ds(g2 * 16, 16)] = lgv
                return jnp.maximum(mxc, lgv)

            mx = pl.loop(0, ne // 16, init_carry=mx)(grp)
            pltpu.sync_copy(lgblk.at[pl.ds(0, ne)],
                            lg_ref.at[pl.ds(gbase, ne)])
            return mx

        @pl.loop(0, NG, init_carry=jnp.full((16,), -jnp.inf, jnp.float32))
        def mx_a(gi, mx):
            return logits_group(tbase + gi * G, [C] * K, mx)

        mxt = logits_group(tbase + NG * G, [C, C, 80], mx_a)

        maxb[...] = mxt
        pltpu.sync_copy(maxb, smx.at[pl.ds(sid * 16, 16)])
        plsc.subcore_barrier()
        pltpu.sync_copy(smx, mxallb)
        gv = jnp.full((16,), -jnp.inf, jnp.float32)
        for s in range(NS):
            gv = jnp.maximum(gv, mxallb[pl.ds(s * 16, 16)])
        g = gv[0]
        for j in range(1, 16):
            g = jnp.maximum(g, gv[j])
        gs = jnp.broadcast_to(g, (16,))
        pltpu.sync_copy(maxb, gmax_ref.at[pl.ds(wid * 16, 16)])

        def accum_group(gbase, sizes, dbufs):
            ne = sum(sizes)
            waits = [
                pltpu.async_copy(src_ref.at[pl.ds(gbase, ne)],
                                 srcblk.at[pl.ds(0, ne)], sem),
                pltpu.async_copy(lg_ref.at[pl.ds(gbase, ne)],
                                 lgblk.at[pl.ds(0, ne)], sem),
            ]
            off = 0
            for sz, db in zip(sizes, dbufs):
                waits.append(pltpu.async_copy(
                    dst_ref.at[pl.ds(gbase + off, sz)], db, sem))
                off += sz
            for wv in waits:
                wv.wait()
            waits = []
            off = 0
            for sz in sizes:
                waits.append(pltpu.async_copy(
                    xl_ref.at[srcblk.at[pl.ds(off, sz)]],
                    xlb.at[pl.ds(off, sz)], sem))
                off += sz
            for wv in waits:
                wv.wait()

            scat = []
            off = 0
            for sz, db in zip(sizes, dbufs):
                base = off

                @pl.loop(0, sz // 16)
                def _(g2):
                    av = jnp.exp(lgblk[pl.ds(base + g2 * 16, 16)] - gs)
                    for j in range(16):
                        e = base + g2 * 16 + j
                        ab = jnp.broadcast_to(av[j], (16,))
                        rowsb[e] = jnp.where(it == 15, ab, xlb[e] * ab)

                scat.append(pltpu.async_copy(
                    rowsb.at[pl.ds(off, sz)], acc.at[db], ssem, add=True))
                off += sz
            for wv in scat:
                wv.wait()

        @pl.loop(0, NG)
        def _(gi):
            accum_group(tbase + gi * G, [C] * K, dstbs)

        accum_group(tbase + NG * G, [C, C, 80], [dstb0, dstb1, dstb_t])

        plsc.subcore_barrier()
        row0 = sid * RPT
        for k in range(NZ):
            pltpu.sync_copy(acc.at[pl.ds(row0 + k * RZ, RZ)], bounce)
            pltpu.sync_copy(bounce, part_ref.at[cid,
                                                pl.ds(row0 + k * RZ, RZ)])

    mesh = plsc.VectorSubcoreMesh(core_axis_name="c", subcore_axis_name="s")
    return pl.kernel(
        body,
        out_type=[jax.ShapeDtypeStruct((NC, NPAD, W), jnp.float32),
                  jax.ShapeDtypeStruct((NW * 16,), jnp.float32),
                  jax.ShapeDtypeStruct((E,), jnp.float32)],
        mesh=mesh,
        compiler_params=_CPARAMS,
        scratch_types=[
            pltpu.VMEM((G,), jnp.int32),
            pltpu.VMEM((G,), jnp.int32),
            pltpu.VMEM((G,), jnp.float32),
            pltpu.VMEM((G, W), jnp.float32),
            pltpu.VMEM((G, W), jnp.float32),
            pltpu.VMEM((G, W), jnp.float32),
            pltpu.VMEM((C,), jnp.int32),
            pltpu.VMEM((C,), jnp.int32),
            pltpu.VMEM((C,), jnp.int32),
            pltpu.VMEM((C,), jnp.int32),
            pltpu.VMEM((80,), jnp.int32),
            pltpu.VMEM((16,), jnp.float32),
            pltpu.VMEM((16,), jnp.float32),
            pltpu.VMEM((NS * 16,), jnp.float32),
            pltpu.VMEM((RZ, W), jnp.float32),
            pltpu.VMEM_SHARED((NPAD, W), jnp.float32),
            pltpu.VMEM_SHARED((NS * 16,), jnp.float32),
            pltpu.SemaphoreType.DMA,
            pltpu.SemaphoreType.DMA,
        ],
    )


_SC_LAYERS = {d: _make_sc_layer(d) for d in (8, 2, 15)}


def _ea_all_layers(edge_attr, Ws):
    LF = edge_attr.shape[1]
    B = 8000

    def bodyf(a_ref, *refs):
        w_refs = refs[:6]
        o_refs = refs[6:]
        a = a_ref[...]
        for w_ref, o_ref in zip(w_refs, o_refs):
            o_ref[...] = jnp.dot(a, w_ref[...],
                                 preferred_element_type=jnp.float32)

    return pl.pallas_call(
        bodyf,
        grid=(E // B,),
        in_specs=[pl.BlockSpec((B, LF), lambda i: (i, 0))] +
                 [pl.BlockSpec((LF, W), lambda i: (0, 0))] * 6,
        out_specs=[pl.BlockSpec((B, W), lambda i: (i, 0))] * 6,
        out_shape=[jax.ShapeDtypeStruct((E, W), jnp.float32)] * 6,
    )(edge_attr, *Ws)


def _transform(h, Wl, Wr):
    din = h.shape[1]
    B = 5000

    def bodyf(h_ref, wl_ref, wr_ref, ol_ref, or_ref):
        hv = h_ref[...]
        ol_ref[...] = jnp.dot(hv, wl_ref[...],
                              preferred_element_type=jnp.float32)
        or_ref[...] = jnp.dot(hv, wr_ref[...],
                              preferred_element_type=jnp.float32)

    return pl.pallas_call(
        bodyf,
        grid=(N // B,),
        in_specs=[pl.BlockSpec((B, din), lambda i: (i, 0)),
                  pl.BlockSpec((din, W), lambda i: (0, 0)),
                  pl.BlockSpec((din, W), lambda i: (0, 0))],
        out_specs=[pl.BlockSpec((B, W), lambda i: (i, 0)),
                   pl.BlockSpec((B, W), lambda i: (i, 0))],
        out_shape=[jax.ShapeDtypeStruct((N, W), jnp.float32),
                   jax.ShapeDtypeStruct((N, W), jnp.float32)],
    )(h, Wl, Wr)


def _combine(part, w2, bias, dout, relu, Wl=None, Wr=None):
    B = 5000
    with_mm = Wl is not None

    def bodyf(part_ref, w_ref, b_ref, *refs):
        w0 = w_ref[0, 0]
        w1 = w_ref[0, 1]
        num = part_ref[0, :, :dout] * w0 + part_ref[1, :, :dout] * w1
        den = part_ref[0, :, 15] * w0 + part_ref[1, :, 15] * w1
        h = num / (den[:, None] + 1e-16) + b_ref[0]
        if relu:
            h = jnp.maximum(h, 0.0)
        if with_mm:
            wl_ref, wr_ref, ol_ref, or_ref = refs
            ol_ref[...] = jnp.dot(h, wl_ref[...],
                                  preferred_element_type=jnp.float32)
            or_ref[...] = jnp.dot(h, wr_ref[...],
                                  preferred_element_type=jnp.float32)
        else:
            refs[0][...] = h

    in_specs = [pl.BlockSpec((NC, B, W), lambda i: (0, i, 0)),
                pl.BlockSpec((1, NC), lambda i: (0, 0)),
                pl.BlockSpec((1, dout), lambda i: (0, 0))]
    args = [part, w2, bias.reshape(1, dout)]
    if with_mm:
        in_specs += [pl.BlockSpec((dout, W), lambda i: (0, 0)),
                     pl.BlockSpec((dout, W), lambda i: (0, 0))]
        args += [Wl, Wr]
        out_specs = [pl.BlockSpec((B, W), lambda i: (i, 0)),
                     pl.BlockSpec((B, W), lambda i: (i, 0))]
        out_shape = [jax.ShapeDtypeStruct((N, W), jnp.float32),
                     jax.ShapeDtypeStruct((N, W), jnp.float32)]
    else:
        out_specs = [pl.BlockSpec((B, dout), lambda i: (i, 0))]
        out_shape = [jax.ShapeDtypeStruct((N, dout), jnp.float32)]

    return pl.pallas_call(
        bodyf,
        grid=(N // B,),
        in_specs=in_specs,
        out_specs=out_specs,
        out_shape=out_shape,
    )(*args)


def kernel(x, edge_index, edge_attr, params):
    src = edge_index[0].astype(jnp.int32)
    dst = edge_index[1].astype(jnp.int32)

    EAs = _ea_all_layers(edge_attr, [_pad16(p["We"]) for p in params])

    xl, xr = _transform(x, _pad16(params[0]["Wl"]), _pad16(params[0]["Wr"]))

    zeros = jnp.zeros((RPT, W), jnp.float32)
    out = None
    for i, p in enumerate(params):
        dout = _DOUTS[i]
        attp = jnp.zeros((16,), jnp.float32).at[:dout].set(p["att"])
        part, gmax, _ = _SC_LAYERS[dout](src, dst, EAs[i], xl, xr, zeros,
                                        attp)
        g_sc = jnp.max(gmax.reshape(NC, NS * 16), axis=1)
        w2 = jnp.exp(g_sc - jnp.max(g_sc)).reshape(1, NC)
        relu = i not in (2, 5)
        if i < 5:
            xl, xr = _combine(part, w2, p["b"], dout, relu,
                              _pad16(params[i + 1]["Wl"]),
                              _pad16(params[i + 1]["Wr"]))
        else:
            out = _combine(part, w2, p["b"], dout, relu)[0]
    return out
